# Initial kernel scaffold; baseline (speedup 1.0000x reference)
import jax
import jax.numpy as jnp
from jax.experimental import pallas as pl


def kernel(row_idx, col_idx, data_tensor, num_ids, embeddings):
    raise NotImplementedError("write your pallas kernel here")


# speedup vs baseline: 1.6959x; 1.6959x over previous
import functools

import jax
import jax.numpy as jnp
from jax import lax
from jax.experimental import pallas as pl
from jax.experimental.pallas import tpu as pltpu
from jax.experimental.pallas import tpu_sc as plsc

NUM_IDS = 16384
NUM_ITEMS = 1000000
EMB = 64
NNZ = 819200
KPROB = 0.8

NC = 2
NS = 16
NW = NC * NS
CHUNK = 128
EPW = NNZ // NW
NCHUNK = EPW // CHUNK
ROWS_PER_TILE = NUM_IDS // NS
LANES = 16


def _sc_body(col_hbm, row_hbm, val16_hbm, emb_hbm, out_hbm,
             colv, rowv, valx, rows, zbuf, acc, gsem):
    cid = lax.axis_index("c")
    sid = lax.axis_index("s")
    wid = sid * NC + cid

    zero = jnp.zeros((LANES,), jnp.float32)
    for e in range(CHUNK):
        for j in range(EMB // LANES):
            zbuf[e, pl.ds(j * LANES, LANES)] = zero
    for k in range(ROWS_PER_TILE // CHUNK):
        pltpu.sync_copy(zbuf, acc.at[pl.ds(sid * ROWS_PER_TILE + k * CHUNK, CHUNK)])
    plsc.subcore_barrier()

    def chunk_body(g, carry):
        off = pl.multiple_of(wid * EPW + g * CHUNK, CHUNK)
        pltpu.sync_copy(col_hbm.at[pl.ds(off, CHUNK)], colv)
        pltpu.sync_copy(row_hbm.at[pl.ds(off, CHUNK)], rowv)
        pltpu.sync_copy(val16_hbm.at[pl.ds(off, CHUNK)], valx)
        pltpu.async_copy(emb_hbm.at[colv], rows, gsem).wait()
        for e in range(CHUNK):
            sv = valx[e, :]
            for j in range(EMB // LANES):
                sl = pl.ds(j * LANES, LANES)
                rows[e, sl] = rows[e, sl] * sv
        pltpu.sync_copy(rows, acc.at[rowv], add=True)
        return carry

    lax.fori_loop(0, NCHUNK, chunk_body, 0)
    plsc.subcore_barrier()

    for k in range(ROWS_PER_TILE // CHUNK):
        r0 = sid * ROWS_PER_TILE + k * CHUNK
        pltpu.sync_copy(acc.at[pl.ds(r0, CHUNK)], out_hbm.at[cid, pl.ds(r0, CHUNK)])


@jax.jit
def _sc_spmm(col_idx, row_idx, val16, embeddings):
    mesh = plsc.VectorSubcoreMesh(core_axis_name="c", subcore_axis_name="s")
    f = pl.kernel(
        _sc_body,
        out_type=jax.ShapeDtypeStruct((NC, NUM_IDS, EMB), jnp.float32),
        mesh=mesh,
        scratch_types=[
            pltpu.VMEM((CHUNK,), jnp.int32),
            pltpu.VMEM((CHUNK,), jnp.int32),
            pltpu.VMEM((CHUNK, LANES), jnp.float32),
            pltpu.VMEM((CHUNK, EMB), jnp.float32),
            pltpu.VMEM((CHUNK, EMB), jnp.float32),
            pltpu.VMEM_SHARED((NUM_IDS, EMB), jnp.float32),
            pltpu.SemaphoreType.DMA,
        ],
        compiler_params=pltpu.CompilerParams(use_tc_tiling_on_sc=False),
    )
    return f(col_idx, row_idx, val16, embeddings)


def _add_body(p_ref, o_ref):
    o_ref[...] = p_ref[0] + p_ref[1]


@jax.jit
def _combine(partials):
    grid = 16
    rows = NUM_IDS // grid
    return pl.pallas_call(
        _add_body,
        out_shape=jax.ShapeDtypeStruct((NUM_IDS, EMB), jnp.float32),
        grid=(grid,),
        in_specs=[pl.BlockSpec((NC, rows, EMB), lambda i: (0, i, 0))],
        out_specs=pl.BlockSpec((rows, EMB), lambda i: (i, 0)),
    )(partials)


def kernel(row_idx, col_idx, data_tensor, num_ids, embeddings):
    mkey = jax.random.key(42)
    mask = jnp.floor(jax.random.uniform(mkey, data_tensor.shape) + KPROB).astype(bool)
    val = jnp.where(mask, data_tensor * (1.0 / KPROB), 0.0)
    val16 = jnp.broadcast_to(val[:, None], (NNZ, LANES))
    partials = _sc_spmm(col_idx.astype(jnp.int32), row_idx.astype(jnp.int32),
                        val16, embeddings)
    return _combine(partials)


# speedup vs baseline: 2.4376x; 1.4373x over previous
import functools

import jax
import jax.numpy as jnp
from jax import lax
from jax.experimental import pallas as pl
from jax.experimental.pallas import tpu as pltpu
from jax.experimental.pallas import tpu_sc as plsc

NUM_IDS = 16384
NUM_ITEMS = 1000000
EMB = 64
NNZ = 819200
KPROB = 0.8

NC = 2
NS = 16
NW = NC * NS
CHUNK = 128
EPW = NNZ // NW
NCHUNK = EPW // CHUNK
NB = 4
ROWS_PER_TILE = NUM_IDS // NS
LANES = 16


def _scale_rows(rows, valx):
    def entry(e, carry):
        for u in range(8):
            sv = valx[e + u, :]
            for j in range(EMB // LANES):
                sl = pl.ds(j * LANES, LANES)
                rows[e + u, sl] = rows[e + u, sl] * sv
        return carry

    lax.fori_loop(0, CHUNK // 8, lambda t, c: entry(t * 8, c), 0)


def _sc_body(col_hbm, row_hbm, val_hbm, emb_hbm, out_hbm, *refs):
    colv = refs[0:NB]
    rowv = refs[NB:2 * NB]
    valx = refs[2 * NB:3 * NB]
    rows = refs[3 * NB:4 * NB]
    acc = refs[4 * NB]
    csem = refs[4 * NB + 1:4 * NB + 1 + NB]
    rsem = refs[4 * NB + 1 + NB:4 * NB + 1 + 2 * NB]
    vsem = refs[4 * NB + 1 + 2 * NB:4 * NB + 1 + 3 * NB]
    gsem = refs[4 * NB + 1 + 3 * NB:4 * NB + 1 + 4 * NB]
    ssem = refs[4 * NB + 1 + 4 * NB:4 * NB + 1 + 5 * NB]

    cid = lax.axis_index("c")
    sid = lax.axis_index("s")
    wid = sid * NC + cid

    zero = jnp.zeros((LANES,), jnp.float32)
    for e in range(CHUNK):
        for j in range(EMB // LANES):
            rows[0][e, pl.ds(j * LANES, LANES)] = zero
    for k in range(ROWS_PER_TILE // CHUNK):
        pltpu.sync_copy(rows[0],
                        acc.at[pl.ds(sid * ROWS_PER_TILE + k * CHUNK, CHUNK)])
    plsc.subcore_barrier()

    def fire_loads(g, b):
        pltpu.async_copy(col_hbm.at[wid, g], colv[b], csem[b])
        pltpu.async_copy(row_hbm.at[wid, g], rowv[b], rsem[b])
        pltpu.async_copy(val_hbm.at[wid, g], valx[b], vsem[b])

    def fire_gather(g, b):
        pltpu.make_async_copy(col_hbm.at[wid, g], colv[b], csem[b]).wait()
        pltpu.async_copy(emb_hbm.at[colv[b]], rows[b], gsem[b])

    def wait_scatter(b):
        pltpu.make_async_copy(rows[b], acc.at[rowv[b]], ssem[b]).wait()

    fire_loads(0, 0)
    fire_loads(1, 1)
    fire_gather(0, 0)

    def quad_body(h, carry):
        g0 = 4 * h
        for i in range(NB):
            g = g0 + i
            bp = (i + 2) % NB
            if i < 2:
                @pl.when(h >= 1)
                def _():
                    wait_scatter(bp)
                fire_loads(g + 2, bp)
            else:
                wait_scatter(bp)

                @pl.when(h < NCHUNK // NB - 1)
                def _():
                    fire_loads(g + 2, bp)
            if i < NB - 1:
                fire_gather(g + 1, (i + 1) % NB)
            else:
                @pl.when(h < NCHUNK // NB - 1)
                def _():
                    fire_gather(g + 1, (i + 1) % NB)
            pltpu.make_async_copy(emb_hbm.at[colv[i]], rows[i], gsem[i]).wait()
            pltpu.make_async_copy(val_hbm.at[wid, g], valx[i], vsem[i]).wait()
            _scale_rows(rows[i], valx[i])
            pltpu.make_async_copy(row_hbm.at[wid, g], rowv[i], rsem[i]).wait()
            pltpu.async_copy(rows[i], acc.at[rowv[i]], ssem[i], add=True)
        return carry

    lax.fori_loop(0, NCHUNK // NB, quad_body, 0)
    wait_scatter(2)
    wait_scatter(3)
    plsc.subcore_barrier()

    for k in range(ROWS_PER_TILE // CHUNK):
        r0 = sid * ROWS_PER_TILE + k * CHUNK
        pltpu.sync_copy(acc.at[pl.ds(r0, CHUNK)], out_hbm.at[cid, pl.ds(r0, CHUNK)])


@jax.jit
def _sc_spmm(col3d, row3d, val4d, embeddings):
    mesh = plsc.VectorSubcoreMesh(core_axis_name="c", subcore_axis_name="s")
    f = pl.kernel(
        _sc_body,
        out_type=jax.ShapeDtypeStruct((NC, NUM_IDS, EMB), jnp.float32),
        mesh=mesh,
        scratch_types=(
            [pltpu.VMEM((CHUNK,), jnp.int32) for _ in range(NB)]
            + [pltpu.VMEM((CHUNK,), jnp.int32) for _ in range(NB)]
            + [pltpu.VMEM((CHUNK, LANES), jnp.float32) for _ in range(NB)]
            + [pltpu.VMEM((CHUNK, EMB), jnp.float32) for _ in range(NB)]
            + [pltpu.VMEM_SHARED((NUM_IDS, EMB), jnp.float32)]
            + [pltpu.SemaphoreType.DMA for _ in range(5 * NB)]
        ),
        compiler_params=pltpu.CompilerParams(use_tc_tiling_on_sc=False),
    )
    return f(col3d, row3d, val4d, embeddings)


def _add_body(p_ref, o_ref):
    o_ref[...] = p_ref[0] + p_ref[1]


@jax.jit
def _combine(partials):
    grid = 16
    rows = NUM_IDS // grid
    return pl.pallas_call(
        _add_body,
        out_shape=jax.ShapeDtypeStruct((NUM_IDS, EMB), jnp.float32),
        grid=(grid,),
        in_specs=[pl.BlockSpec((NC, rows, EMB), lambda i: (0, i, 0))],
        out_specs=pl.BlockSpec((rows, EMB), lambda i: (i, 0)),
    )(partials)


def kernel(row_idx, col_idx, data_tensor, num_ids, embeddings):
    mkey = jax.random.key(42)
    mask = jnp.floor(jax.random.uniform(mkey, data_tensor.shape) + KPROB).astype(bool)
    val = jnp.where(mask, data_tensor * (1.0 / KPROB), 0.0)
    val4d = jnp.broadcast_to(
        val.reshape(NW, NCHUNK, CHUNK, 1), (NW, NCHUNK, CHUNK, LANES))
    col3d = col_idx.astype(jnp.int32).reshape(NW, NCHUNK, CHUNK)
    row3d = row_idx.astype(jnp.int32).reshape(NW, NCHUNK, CHUNK)
    partials = _sc_spmm(col3d, row3d, val4d, embeddings)
    return _combine(partials)


# speedup vs baseline: 2.9460x; 1.2086x over previous
import functools

import jax
import jax.numpy as jnp
from jax import lax
from jax.experimental import pallas as pl
from jax.experimental.pallas import tpu as pltpu
from jax.experimental.pallas import tpu_sc as plsc

NUM_IDS = 16384
NUM_ITEMS = 1000000
EMB = 64
NNZ = 819200
KPROB = 0.8

NC = 2
NS = 16
NW = NC * NS
CHUNK = 128
EPW = NNZ // NW
NCHUNK = EPW // CHUNK
NB = 4
ROWS_PER_TILE = NUM_IDS // NS
LANES = 16


def _splat(vec, i):
    idx = jnp.full((LANES,), i, jnp.int32)
    return lax.gather(
        vec, idx[:, None],
        dimension_numbers=lax.GatherDimensionNumbers(
            offset_dims=(), collapsed_slice_dims=(0,), start_index_map=(0,)),
        slice_sizes=(1,), mode=lax.GatherScatterMode.PROMISE_IN_BOUNDS)


def _scale_rows(rows, valx):
    def group(e0, carry):
        vvec = valx[pl.ds(e0, LANES)]
        for i in range(LANES):
            sv = _splat(vvec, i)
            for j in range(EMB // LANES):
                sl = pl.ds(j * LANES, LANES)
                rows[e0 + i, sl] = rows[e0 + i, sl] * sv
        return carry

    lax.fori_loop(0, CHUNK // LANES, lambda t, c: group(t * LANES, c), 0)


def _sc_body(col_hbm, row_hbm, val_hbm, emb_hbm, out_hbm, *refs):
    colv = refs[0:NB]
    rowv = refs[NB:2 * NB]
    valx = refs[2 * NB:3 * NB]
    rows = refs[3 * NB:4 * NB]
    acc = refs[4 * NB]
    csem = refs[4 * NB + 1:4 * NB + 1 + NB]
    rsem = refs[4 * NB + 1 + NB:4 * NB + 1 + 2 * NB]
    vsem = refs[4 * NB + 1 + 2 * NB:4 * NB + 1 + 3 * NB]
    gsem = refs[4 * NB + 1 + 3 * NB:4 * NB + 1 + 4 * NB]
    ssem = refs[4 * NB + 1 + 4 * NB:4 * NB + 1 + 5 * NB]

    cid = lax.axis_index("c")
    sid = lax.axis_index("s")
    wid = sid * NC + cid
    base = wid * EPW

    zero = jnp.zeros((LANES,), jnp.float32)
    for e in range(CHUNK):
        for j in range(EMB // LANES):
            rows[0][e, pl.ds(j * LANES, LANES)] = zero
    for k in range(ROWS_PER_TILE // CHUNK):
        pltpu.sync_copy(rows[0],
                        acc.at[pl.ds(sid * ROWS_PER_TILE + k * CHUNK, CHUNK)])
    plsc.subcore_barrier()

    def fire_loads(g, b):
        off = pl.multiple_of(base + g * CHUNK, CHUNK)
        pltpu.async_copy(col_hbm.at[pl.ds(off, CHUNK)], colv[b], csem[b])
        pltpu.async_copy(row_hbm.at[pl.ds(off, CHUNK)], rowv[b], rsem[b])
        pltpu.async_copy(val_hbm.at[pl.ds(off, CHUNK)], valx[b], vsem[b])

    def fire_gather(g, b):
        off = pl.multiple_of(base + g * CHUNK, CHUNK)
        pltpu.make_async_copy(col_hbm.at[pl.ds(off, CHUNK)], colv[b],
                              csem[b]).wait()
        pltpu.async_copy(emb_hbm.at[colv[b]], rows[b], gsem[b])

    def wait_scatter(b):
        pltpu.make_async_copy(rows[b], acc.at[rowv[b]], ssem[b]).wait()

    fire_loads(0, 0)
    fire_loads(1, 1)
    fire_gather(0, 0)

    def quad_body(h, carry):
        g0 = 4 * h
        for i in range(NB):
            g = g0 + i
            bp = (i + 2) % NB
            if i < 2:
                @pl.when(h >= 1)
                def _():
                    wait_scatter(bp)
                fire_loads(g + 2, bp)
            else:
                wait_scatter(bp)

                @pl.when(h < NCHUNK // NB - 1)
                def _():
                    fire_loads(g + 2, bp)
            if i < NB - 1:
                fire_gather(g + 1, (i + 1) % NB)
            else:
                @pl.when(h < NCHUNK // NB - 1)
                def _():
                    fire_gather(g + 1, (i + 1) % NB)
            off = pl.multiple_of(base + g * CHUNK, CHUNK)
            pltpu.make_async_copy(emb_hbm.at[colv[i]], rows[i], gsem[i]).wait()
            pltpu.make_async_copy(val_hbm.at[pl.ds(off, CHUNK)], valx[i],
                                  vsem[i]).wait()
            _scale_rows(rows[i], valx[i])
            pltpu.make_async_copy(row_hbm.at[pl.ds(off, CHUNK)], rowv[i],
                                  rsem[i]).wait()
            pltpu.async_copy(rows[i], acc.at[rowv[i]], ssem[i], add=True)
        return carry

    lax.fori_loop(0, NCHUNK // NB, quad_body, 0)
    wait_scatter(2)
    wait_scatter(3)
    plsc.subcore_barrier()

    for k in range(ROWS_PER_TILE // CHUNK):
        r0 = sid * ROWS_PER_TILE + k * CHUNK
        pltpu.sync_copy(acc.at[pl.ds(r0, CHUNK)], out_hbm.at[cid, pl.ds(r0, CHUNK)])


@jax.jit
def _sc_spmm(col2, row_idx, val, embp):
    emb2m = embp.reshape(2 * NUM_ITEMS, EMB)
    mesh = plsc.VectorSubcoreMesh(core_axis_name="c", subcore_axis_name="s")
    f = pl.kernel(
        _sc_body,
        out_type=jax.ShapeDtypeStruct((NC, NUM_IDS, EMB), jnp.float32),
        mesh=mesh,
        scratch_types=(
            [pltpu.VMEM((CHUNK,), jnp.int32) for _ in range(NB)]
            + [pltpu.VMEM((CHUNK,), jnp.int32) for _ in range(NB)]
            + [pltpu.VMEM((CHUNK,), jnp.float32) for _ in range(NB)]
            + [pltpu.VMEM((CHUNK, EMB), jnp.float32) for _ in range(NB)]
            + [pltpu.VMEM_SHARED((NUM_IDS, EMB), jnp.float32)]
            + [pltpu.SemaphoreType.DMA for _ in range(5 * NB)]
        ),
        compiler_params=pltpu.CompilerParams(use_tc_tiling_on_sc=False),
    )
    return f(col2, row_idx, val, emb2m).reshape(-1)


def _add_body(a_ref, b_ref, o_ref):
    o_ref[...] = a_ref[...] + b_ref[...]


HALF = NUM_IDS * EMB
BLK = 65536


@jax.jit
def _combine(pflat):
    return pl.pallas_call(
        _add_body,
        out_shape=jax.ShapeDtypeStruct((HALF,), jnp.float32),
        grid=(HALF // BLK,),
        in_specs=[pl.BlockSpec((BLK,), lambda i: (i,)),
                  pl.BlockSpec((BLK,), lambda i: (i + HALF // BLK,))],
        out_specs=pl.BlockSpec((BLK,), lambda i: (i,)),
    )(pflat, pflat)


def kernel(row_idx, col_idx, data_tensor, num_ids, embeddings):
    mkey = jax.random.key(42)
    mask = jnp.floor(jax.random.uniform(mkey, data_tensor.shape) + KPROB).astype(bool)
    val = jnp.where(mask, data_tensor * (1.0 / KPROB), 0.0)
    col2 = (col_idx * 2).astype(jnp.int32)
    row32 = row_idx.astype(jnp.int32)
    embp = jnp.pad(embeddings, ((0, 0), (0, EMB)))
    pflat = _sc_spmm(col2, row32, val, embp)
    return _combine(pflat).reshape(NUM_IDS, EMB)


# speedup vs baseline: 3.9851x; 1.3527x over previous
import functools

import jax
import jax.numpy as jnp
from jax import lax
from jax.experimental import pallas as pl
from jax.experimental.pallas import tpu as pltpu
from jax.experimental.pallas import tpu_sc as plsc

NUM_IDS = 16384
NUM_ITEMS = 1000000
EMB = 64
NNZ = 819200
KPROB = 0.8

NC = 2
NS = 16
NW = NC * NS
CHUNK = 128
EPW = NNZ // NW
NCHUNK = EPW // CHUNK
NB = 4
ROWS_PER_TILE = NUM_IDS // NS
LANES = 16


def _splat(vec, i):
    idx = jnp.full((LANES,), i, jnp.int32)
    return lax.gather(
        vec, idx[:, None],
        dimension_numbers=lax.GatherDimensionNumbers(
            offset_dims=(), collapsed_slice_dims=(0,), start_index_map=(0,)),
        slice_sizes=(1,), mode=lax.GatherScatterMode.PROMISE_IN_BOUNDS)


def _scale_rows(rows, valx):
    @plsc.parallel_loop(0, CHUNK, step=LANES, unroll=2)
    def _(e0):
        vvec = valx[pl.ds(e0, LANES)]
        svs = [_splat(vvec, i) for i in range(LANES)]
        for i in range(LANES):
            for j in range(EMB // LANES):
                sl = pl.ds(j * LANES, LANES)
                rows[e0 + i, sl] = rows[e0 + i, sl] * svs[i]


def _sc_body(col_hbm, row_hbm, val_hbm, emb_hbm, out_hbm, *refs):
    colv = refs[0:NB]
    rowv = refs[NB:2 * NB]
    valx = refs[2 * NB:3 * NB]
    rows = refs[3 * NB:4 * NB]
    acc = refs[4 * NB]
    csem = refs[4 * NB + 1:4 * NB + 1 + NB]
    rsem = refs[4 * NB + 1 + NB:4 * NB + 1 + 2 * NB]
    vsem = refs[4 * NB + 1 + 2 * NB:4 * NB + 1 + 3 * NB]
    gsem = refs[4 * NB + 1 + 3 * NB:4 * NB + 1 + 4 * NB]
    ssem = refs[4 * NB + 1 + 4 * NB:4 * NB + 1 + 5 * NB]

    cid = lax.axis_index("c")
    sid = lax.axis_index("s")
    wid = sid * NC + cid
    base = wid * EPW

    zero = jnp.zeros((LANES,), jnp.float32)
    for e in range(CHUNK):
        for j in range(EMB // LANES):
            rows[0][e, pl.ds(j * LANES, LANES)] = zero
    for k in range(ROWS_PER_TILE // CHUNK):
        pltpu.sync_copy(rows[0],
                        acc.at[pl.ds(sid * ROWS_PER_TILE + k * CHUNK, CHUNK)])
    plsc.subcore_barrier()

    def fire_loads(g, b):
        off = pl.multiple_of(base + g * CHUNK, CHUNK)
        pltpu.async_copy(col_hbm.at[pl.ds(off, CHUNK)], colv[b], csem[b])
        pltpu.async_copy(row_hbm.at[pl.ds(off, CHUNK)], rowv[b], rsem[b])
        pltpu.async_copy(val_hbm.at[pl.ds(off, CHUNK)], valx[b], vsem[b])

    def fire_gather(g, b):
        off = pl.multiple_of(base + g * CHUNK, CHUNK)
        pltpu.make_async_copy(col_hbm.at[pl.ds(off, CHUNK)], colv[b],
                              csem[b]).wait()
        pltpu.async_copy(emb_hbm.at[colv[b]], rows[b], gsem[b])

    def wait_scatter(b):
        pltpu.make_async_copy(rows[b], acc.at[rowv[b]], ssem[b]).wait()

    fire_loads(0, 0)
    fire_loads(1, 1)
    fire_gather(0, 0)

    def quad_body(h, carry):
        g0 = 4 * h
        for i in range(NB):
            g = g0 + i
            bp = (i + 2) % NB
            if i < 2:
                @pl.when(h >= 1)
                def _():
                    wait_scatter(bp)
                fire_loads(g + 2, bp)
            else:
                wait_scatter(bp)

                @pl.when(h < NCHUNK // NB - 1)
                def _():
                    fire_loads(g + 2, bp)
            if i < NB - 1:
                fire_gather(g + 1, (i + 1) % NB)
            else:
                @pl.when(h < NCHUNK // NB - 1)
                def _():
                    fire_gather(g + 1, (i + 1) % NB)
            off = pl.multiple_of(base + g * CHUNK, CHUNK)
            pltpu.make_async_copy(emb_hbm.at[colv[i]], rows[i], gsem[i]).wait()
            pltpu.make_async_copy(val_hbm.at[pl.ds(off, CHUNK)], valx[i],
                                  vsem[i]).wait()
            _scale_rows(rows[i], valx[i])
            pltpu.make_async_copy(row_hbm.at[pl.ds(off, CHUNK)], rowv[i],
                                  rsem[i]).wait()
            pltpu.async_copy(rows[i], acc.at[rowv[i]], ssem[i], add=True)
        return carry

    lax.fori_loop(0, NCHUNK // NB, quad_body, 0)
    wait_scatter(2)
    wait_scatter(3)
    plsc.subcore_barrier()

    for k in range(ROWS_PER_TILE // CHUNK):
        r0 = sid * ROWS_PER_TILE + k * CHUNK
        pltpu.sync_copy(acc.at[pl.ds(r0, CHUNK)], out_hbm.at[cid, pl.ds(r0, CHUNK)])


@jax.jit
def _sc_spmm(col2, row_idx, val, embp):
    emb2m = embp.reshape(2 * NUM_ITEMS, EMB)
    mesh = plsc.VectorSubcoreMesh(core_axis_name="c", subcore_axis_name="s")
    f = pl.kernel(
        _sc_body,
        out_type=jax.ShapeDtypeStruct((NC, NUM_IDS, EMB), jnp.float32),
        mesh=mesh,
        scratch_types=(
            [pltpu.VMEM((CHUNK,), jnp.int32) for _ in range(NB)]
            + [pltpu.VMEM((CHUNK,), jnp.int32) for _ in range(NB)]
            + [pltpu.VMEM((CHUNK,), jnp.float32) for _ in range(NB)]
            + [pltpu.VMEM((CHUNK, EMB), jnp.float32) for _ in range(NB)]
            + [pltpu.VMEM_SHARED((NUM_IDS, EMB), jnp.float32)]
            + [pltpu.SemaphoreType.DMA for _ in range(5 * NB)]
        ),
        compiler_params=pltpu.CompilerParams(use_tc_tiling_on_sc=False),
    )
    return f(col2, row_idx, val, emb2m).reshape(-1)


def _add_body(a_ref, b_ref, o_ref):
    o_ref[...] = a_ref[...] + b_ref[...]


HALF = NUM_IDS * EMB
BLK = 65536


@jax.jit
def _combine(pflat):
    return pl.pallas_call(
        _add_body,
        out_shape=jax.ShapeDtypeStruct((HALF,), jnp.float32),
        grid=(HALF // BLK,),
        in_specs=[pl.BlockSpec((BLK,), lambda i: (i,)),
                  pl.BlockSpec((BLK,), lambda i: (i + HALF // BLK,))],
        out_specs=pl.BlockSpec((BLK,), lambda i: (i,)),
    )(pflat, pflat)


def kernel(row_idx, col_idx, data_tensor, num_ids, embeddings):
    mkey = jax.random.key(42)
    mask = jnp.floor(jax.random.uniform(mkey, data_tensor.shape) + KPROB).astype(bool)
    val = jnp.where(mask, data_tensor * (1.0 / KPROB), 0.0)
    col2 = (col_idx * 2).astype(jnp.int32)
    row32 = row_idx.astype(jnp.int32)
    embp = jnp.pad(embeddings, ((0, 0), (0, EMB)))
    pflat = _sc_spmm(col2, row32, val, embp)
    return _combine(pflat).reshape(NUM_IDS, EMB)


# speedup vs baseline: 4.2992x; 1.0788x over previous
import functools

import jax
import jax.numpy as jnp
from jax import lax
from jax.experimental import pallas as pl
from jax.experimental.pallas import tpu as pltpu
from jax.experimental.pallas import tpu_sc as plsc

NUM_IDS = 16384
NUM_ITEMS = 1000000
EMB = 64
NNZ = 819200
KPROB = 0.8

NC = 2
NS = 16
NW = NC * NS
CHUNK = 128
EPW = NNZ // NW
NCHUNK = EPW // CHUNK
NB = 4
ROWS_PER_TILE = NUM_IDS // NS
LANES = 16


def _splat(vec, i):
    idx = jnp.full((LANES,), i, jnp.int32)
    return lax.gather(
        vec, idx[:, None],
        dimension_numbers=lax.GatherDimensionNumbers(
            offset_dims=(), collapsed_slice_dims=(0,), start_index_map=(0,)),
        slice_sizes=(1,), mode=lax.GatherScatterMode.PROMISE_IN_BOUNDS)


def _scale_rows(rows, valx):
    @plsc.parallel_loop(0, CHUNK, step=LANES, unroll=2)
    def _(e0):
        vvec = valx[pl.ds(e0, LANES)]
        svs = [_splat(vvec, i) for i in range(LANES)]
        for i in range(LANES):
            for j in range(EMB // LANES):
                sl = pl.ds(j * LANES, LANES)
                rows[e0 + i, sl] = rows[e0 + i, sl] * svs[i]


def _sc_body(col_hbm, row_hbm, val_hbm, emb_hbm, out_hbm, *refs):
    colv = refs[0:NB]
    rowv = refs[NB:2 * NB]
    valx = refs[2 * NB:3 * NB]
    rows = refs[3 * NB:4 * NB]
    acc = refs[4 * NB]
    csem = refs[4 * NB + 1:4 * NB + 1 + NB]
    rsem = refs[4 * NB + 1 + NB:4 * NB + 1 + 2 * NB]
    vsem = refs[4 * NB + 1 + 2 * NB:4 * NB + 1 + 3 * NB]
    gsem = refs[4 * NB + 1 + 3 * NB:4 * NB + 1 + 4 * NB]
    ssem = refs[4 * NB + 1 + 4 * NB:4 * NB + 1 + 5 * NB]

    cid = lax.axis_index("c")
    sid = lax.axis_index("s")
    wid = sid * NC + cid
    base = wid * EPW

    zero = jnp.zeros((LANES,), jnp.float32)
    for e in range(CHUNK):
        for j in range(EMB // LANES):
            rows[0][e, pl.ds(j * LANES, LANES)] = zero
    for k in range(ROWS_PER_TILE // CHUNK):
        pltpu.sync_copy(rows[0],
                        acc.at[pl.ds(sid * ROWS_PER_TILE + k * CHUNK, CHUNK)])
    plsc.subcore_barrier()

    def fire_loads(g, b):
        off = pl.multiple_of(base + g * CHUNK, CHUNK)
        pltpu.async_copy(col_hbm.at[pl.ds(off, CHUNK)], colv[b], csem[b])
        pltpu.async_copy(row_hbm.at[pl.ds(off, CHUNK)], rowv[b], rsem[b])
        pltpu.async_copy(val_hbm.at[pl.ds(off, CHUNK)], valx[b], vsem[b])

    def fire_gather(g, b):
        off = pl.multiple_of(base + g * CHUNK, CHUNK)
        pltpu.make_async_copy(col_hbm.at[pl.ds(off, CHUNK)], colv[b],
                              csem[b]).wait()
        pltpu.async_copy(emb_hbm.at[colv[b]], rows[b], gsem[b])

    def wait_scatter(b):
        pltpu.make_async_copy(rows[b], acc.at[rowv[b]], ssem[b]).wait()

    fire_loads(0, 0)
    fire_loads(1, 1)
    fire_gather(0, 0)

    def quad_body(h, carry):
        g0 = 4 * h
        for i in range(NB):
            g = g0 + i
            bp = (i + 2) % NB
            if i < 2:
                @pl.when(h >= 1)
                def _():
                    wait_scatter(bp)
                fire_loads(g + 2, bp)
            else:
                wait_scatter(bp)

                @pl.when(h < NCHUNK // NB - 1)
                def _():
                    fire_loads(g + 2, bp)
            if i < NB - 1:
                fire_gather(g + 1, (i + 1) % NB)
            else:
                @pl.when(h < NCHUNK // NB - 1)
                def _():
                    fire_gather(g + 1, (i + 1) % NB)
            off = pl.multiple_of(base + g * CHUNK, CHUNK)
            pltpu.make_async_copy(emb_hbm.at[colv[i]], rows[i], gsem[i]).wait()
            pltpu.make_async_copy(val_hbm.at[pl.ds(off, CHUNK)], valx[i],
                                  vsem[i]).wait()
            _scale_rows(rows[i], valx[i])
            pltpu.make_async_copy(row_hbm.at[pl.ds(off, CHUNK)], rowv[i],
                                  rsem[i]).wait()
            pltpu.async_copy(rows[i], acc.at[rowv[i]], ssem[i], add=True)
        return carry

    lax.fori_loop(0, NCHUNK // NB, quad_body, 0)
    wait_scatter(2)
    wait_scatter(3)
    plsc.subcore_barrier()

    for k in range(ROWS_PER_TILE // CHUNK):
        r0 = sid * ROWS_PER_TILE + k * CHUNK
        pltpu.sync_copy(acc.at[pl.ds(r0, CHUNK)], out_hbm.at[cid, pl.ds(r0, CHUNK)])


@jax.jit
def _sc_spmm(col2, row_idx, val, embp):
    emb2m = embp.reshape(2 * NUM_ITEMS, EMB)
    mesh = plsc.VectorSubcoreMesh(core_axis_name="c", subcore_axis_name="s")
    f = pl.kernel(
        _sc_body,
        out_type=jax.ShapeDtypeStruct((NC, NUM_IDS, EMB), jnp.float32),
        mesh=mesh,
        scratch_types=(
            [pltpu.VMEM((CHUNK,), jnp.int32) for _ in range(NB)]
            + [pltpu.VMEM((CHUNK,), jnp.int32) for _ in range(NB)]
            + [pltpu.VMEM((CHUNK,), jnp.float32) for _ in range(NB)]
            + [pltpu.VMEM((CHUNK, EMB), jnp.float32) for _ in range(NB)]
            + [pltpu.VMEM_SHARED((NUM_IDS, EMB), jnp.float32)]
            + [pltpu.SemaphoreType.DMA for _ in range(5 * NB)]
        ),
        compiler_params=pltpu.CompilerParams(use_tc_tiling_on_sc=False),
    )
    return f(col2, row_idx, val, emb2m).reshape(-1)


TR = 2048


def _tp_body(t_ref, o_ref):
    y = jnp.swapaxes(t_ref[...], 0, 1)
    o_ref[...] = jnp.concatenate(
        [y, jnp.zeros((y.shape[0], EMB), jnp.float32)], axis=1)


@jax.jit
def _transpose_pad(embt):
    return pl.pallas_call(
        _tp_body,
        out_shape=jax.ShapeDtypeStruct((NUM_ITEMS, 2 * EMB), jnp.float32),
        grid=(pl.cdiv(NUM_ITEMS, TR),),
        in_specs=[pl.BlockSpec((EMB, TR), lambda i: (0, i))],
        out_specs=pl.BlockSpec((TR, 2 * EMB), lambda i: (i, 0)),
    )(embt)


def _add_body(a_ref, b_ref, o_ref):
    o_ref[...] = a_ref[...] + b_ref[...]


HALF = NUM_IDS * EMB
BLK = 65536


@jax.jit
def _combine(pflat):
    return pl.pallas_call(
        _add_body,
        out_shape=jax.ShapeDtypeStruct((HALF,), jnp.float32),
        grid=(HALF // BLK,),
        in_specs=[pl.BlockSpec((BLK,), lambda i: (i,)),
                  pl.BlockSpec((BLK,), lambda i: (i + HALF // BLK,))],
        out_specs=pl.BlockSpec((BLK,), lambda i: (i,)),
    )(pflat, pflat)


def kernel(row_idx, col_idx, data_tensor, num_ids, embeddings):
    mkey = jax.random.key(42)
    mask = jnp.floor(jax.random.uniform(mkey, data_tensor.shape) + KPROB).astype(bool)
    val = jnp.where(mask, data_tensor * (1.0 / KPROB), 0.0)
    col2 = (col_idx * 2).astype(jnp.int32)
    row32 = row_idx.astype(jnp.int32)
    embp = _transpose_pad(embeddings.T)
    pflat = _sc_spmm(col2, row32, val, embp)
    return _combine(pflat).reshape(NUM_IDS, EMB)


# speedup vs baseline: 6.0514x; 1.4076x over previous
import functools

import jax
import jax.numpy as jnp
from jax import lax
from jax.experimental import pallas as pl
from jax.experimental.pallas import tpu as pltpu
from jax.experimental.pallas import tpu_sc as plsc

NUM_IDS = 16384
NUM_ITEMS = 1000000
EMB = 64
NNZ = 819200
KPROB = 0.8

NC = 2
NS = 16
NW = NC * NS
CHUNK = 128
EPW = NNZ // NW
NCHUNK = EPW // CHUNK
NB = 4
ROWS_PER_TILE = NUM_IDS // NS
LANES = 16


def _splat(vec, i):
    idx = jnp.full((LANES,), i, jnp.int32)
    return lax.gather(
        vec, idx[:, None],
        dimension_numbers=lax.GatherDimensionNumbers(
            offset_dims=(), collapsed_slice_dims=(0,), start_index_map=(0,)),
        slice_sizes=(1,), mode=lax.GatherScatterMode.PROMISE_IN_BOUNDS)


def _scale_rows(rows, valx):
    @plsc.parallel_loop(0, CHUNK, step=LANES, unroll=2)
    def _(e0):
        vvec = valx[pl.ds(e0, LANES)]
        svs = [_splat(vvec, i) for i in range(LANES)]
        for i in range(LANES):
            for j in range(EMB // LANES):
                sl = pl.ds(j * LANES, LANES)
                rows[e0 + i, sl] = rows[e0 + i, sl] * svs[i]


def _sc_body(col_hbm, row_hbm, val_hbm, emb_hbm, out_hbm, *refs):
    colv = refs[0:NB]
    rowv = refs[NB:2 * NB]
    valx = refs[2 * NB:3 * NB]
    rows = refs[3 * NB:4 * NB]
    acc = refs[4 * NB]
    csem = refs[4 * NB + 1:4 * NB + 1 + NB]
    rsem = refs[4 * NB + 1 + NB:4 * NB + 1 + 2 * NB]
    vsem = refs[4 * NB + 1 + 2 * NB:4 * NB + 1 + 3 * NB]
    gsem = refs[4 * NB + 1 + 3 * NB:4 * NB + 1 + 4 * NB]
    ssem = refs[4 * NB + 1 + 4 * NB:4 * NB + 1 + 5 * NB]

    cid = lax.axis_index("c")
    sid = lax.axis_index("s")
    wid = sid * NC + cid
    base = wid * EPW

    zero = jnp.zeros((LANES,), jnp.float32)
    for e in range(CHUNK):
        for j in range(EMB // LANES):
            rows[0][e, pl.ds(j * LANES, LANES)] = zero
    for k in range(ROWS_PER_TILE // CHUNK):
        pltpu.sync_copy(rows[0],
                        acc.at[pl.ds(sid * ROWS_PER_TILE + k * CHUNK, CHUNK)])
    plsc.subcore_barrier()

    def fire_loads(g, b):
        off = pl.multiple_of(base + g * CHUNK, CHUNK)
        pltpu.async_copy(col_hbm.at[pl.ds(off, CHUNK)], colv[b], csem[b])
        pltpu.async_copy(row_hbm.at[pl.ds(off, CHUNK)], rowv[b], rsem[b])
        pltpu.async_copy(val_hbm.at[pl.ds(off, CHUNK)], valx[b], vsem[b])

    def fire_gather(g, b):
        off = pl.multiple_of(base + g * CHUNK, CHUNK)
        pltpu.make_async_copy(col_hbm.at[pl.ds(off, CHUNK)], colv[b],
                              csem[b]).wait()
        pltpu.async_copy(emb_hbm.at[colv[b]], rows[b], gsem[b])

    def wait_scatter(b):
        pltpu.make_async_copy(rows[b], acc.at[rowv[b]], ssem[b]).wait()

    fire_loads(0, 0)
    fire_loads(1, 1)
    fire_gather(0, 0)

    def quad_body(h, carry):
        g0 = 4 * h
        for i in range(NB):
            g = g0 + i
            bp = (i + 2) % NB
            if i < 2:
                @pl.when(h >= 1)
                def _():
                    wait_scatter(bp)
                fire_loads(g + 2, bp)
            else:
                wait_scatter(bp)

                @pl.when(h < NCHUNK // NB - 1)
                def _():
                    fire_loads(g + 2, bp)
            if i < NB - 1:
                fire_gather(g + 1, (i + 1) % NB)
            else:
                @pl.when(h < NCHUNK // NB - 1)
                def _():
                    fire_gather(g + 1, (i + 1) % NB)
            off = pl.multiple_of(base + g * CHUNK, CHUNK)
            pltpu.make_async_copy(emb_hbm.at[colv[i]], rows[i], gsem[i]).wait()
            pltpu.make_async_copy(val_hbm.at[pl.ds(off, CHUNK)], valx[i],
                                  vsem[i]).wait()
            _scale_rows(rows[i], valx[i])
            pltpu.make_async_copy(row_hbm.at[pl.ds(off, CHUNK)], rowv[i],
                                  rsem[i]).wait()
            pltpu.async_copy(rows[i], acc.at[rowv[i]], ssem[i], add=True)
        return carry

    lax.fori_loop(0, NCHUNK // NB, quad_body, 0)
    wait_scatter(2)
    wait_scatter(3)
    plsc.subcore_barrier()

    for k in range(ROWS_PER_TILE // CHUNK):
        r0 = sid * ROWS_PER_TILE + k * CHUNK
        pltpu.sync_copy(acc.at[pl.ds(r0, CHUNK)], out_hbm.at[cid, pl.ds(r0, CHUNK)])


@jax.jit
def _sc_spmm(col2, row_idx, val, embp):
    emb2m = embp.reshape(2 * NUM_ITEMS, EMB)
    mesh = plsc.VectorSubcoreMesh(core_axis_name="c", subcore_axis_name="s")
    f = pl.kernel(
        _sc_body,
        out_type=jax.ShapeDtypeStruct((NC, NUM_IDS, EMB), jnp.float32),
        mesh=mesh,
        scratch_types=(
            [pltpu.VMEM((CHUNK,), jnp.int32) for _ in range(NB)]
            + [pltpu.VMEM((CHUNK,), jnp.int32) for _ in range(NB)]
            + [pltpu.VMEM((CHUNK,), jnp.float32) for _ in range(NB)]
            + [pltpu.VMEM((CHUNK, EMB), jnp.float32) for _ in range(NB)]
            + [pltpu.VMEM_SHARED((NUM_IDS, EMB), jnp.float32)]
            + [pltpu.SemaphoreType.DMA for _ in range(5 * NB)]
        ),
        compiler_params=pltpu.CompilerParams(use_tc_tiling_on_sc=False),
    )
    return f(col2, row_idx, val, emb2m).reshape(-1)


TR = 8192


def _tp_body(t_ref, o_ref):
    y = jnp.swapaxes(t_ref[...], 0, 1)
    o_ref[...] = jnp.concatenate(
        [y, jnp.zeros((y.shape[0], EMB), jnp.float32)], axis=1)


@jax.jit
def _transpose_pad(embt):
    return pl.pallas_call(
        _tp_body,
        out_shape=jax.ShapeDtypeStruct((NUM_ITEMS, 2 * EMB), jnp.float32),
        grid=(pl.cdiv(NUM_ITEMS, TR),),
        in_specs=[pl.BlockSpec((EMB, TR), lambda i: (0, i))],
        out_specs=pl.BlockSpec((TR, 2 * EMB), lambda i: (i, 0)),
    )(embt)


def _add_body(a_ref, b_ref, o_ref):
    o_ref[...] = a_ref[...] + b_ref[...]


HALF = NUM_IDS * EMB
BLK = 65536


@jax.jit
def _combine(pflat):
    return pl.pallas_call(
        _add_body,
        out_shape=jax.ShapeDtypeStruct((HALF,), jnp.float32),
        grid=(HALF // BLK,),
        in_specs=[pl.BlockSpec((BLK,), lambda i: (i,)),
                  pl.BlockSpec((BLK,), lambda i: (i + HALF // BLK,))],
        out_specs=pl.BlockSpec((BLK,), lambda i: (i,)),
    )(pflat, pflat)


def kernel(row_idx, col_idx, data_tensor, num_ids, embeddings):
    mkey = jax.random.key(42)
    mask = jnp.floor(jax.random.uniform(mkey, data_tensor.shape) + KPROB).astype(bool)
    val = jnp.where(mask, data_tensor * (1.0 / KPROB), 0.0)
    col2 = (col_idx * 2).astype(jnp.int32)
    row32 = row_idx.astype(jnp.int32)
    embp = _transpose_pad(embeddings.T)
    pflat = _sc_spmm(col2, row32, val, embp)
    return _combine(pflat).reshape(NUM_IDS, EMB)


# speedup vs baseline: 6.3153x; 1.0436x over previous
import functools

import jax
import jax.numpy as jnp
from jax import lax
from jax.experimental import pallas as pl
from jax.experimental.pallas import tpu as pltpu
from jax.experimental.pallas import tpu_sc as plsc

NUM_IDS = 16384
NUM_ITEMS = 1000000
EMB = 64
NNZ = 819200
KPROB = 0.8

NC = 2
NS = 16
NW = NC * NS
CHUNK = 128
EPW = NNZ // NW
NCHUNK = EPW // CHUNK
NB = 4
ROWS_PER_TILE = NUM_IDS // NS
LANES = 16


def _splat(vec, i):
    idx = jnp.full((LANES,), i, jnp.int32)
    return lax.gather(
        vec, idx[:, None],
        dimension_numbers=lax.GatherDimensionNumbers(
            offset_dims=(), collapsed_slice_dims=(0,), start_index_map=(0,)),
        slice_sizes=(1,), mode=lax.GatherScatterMode.PROMISE_IN_BOUNDS)


def _scale_rows(rows, valx):
    @plsc.parallel_loop(0, CHUNK, step=LANES, unroll=2)
    def _(e0):
        vvec = valx[pl.ds(e0, LANES)]
        svs = [_splat(vvec, i) for i in range(LANES)]
        for i in range(LANES):
            for j in range(EMB // LANES):
                sl = pl.ds(j * LANES, LANES)
                rows[e0 + i, sl] = rows[e0 + i, sl] * svs[i]


def _sc_body(col_hbm, row_hbm, val_hbm, emb_hbm, out_hbm, *refs):
    colv = refs[0:NB]
    rowv = refs[NB:2 * NB]
    valx = refs[2 * NB:3 * NB]
    rows = refs[3 * NB:4 * NB]
    acc = refs[4 * NB]
    csem = refs[4 * NB + 1:4 * NB + 1 + NB]
    rsem = refs[4 * NB + 1 + NB:4 * NB + 1 + 2 * NB]
    vsem = refs[4 * NB + 1 + 2 * NB:4 * NB + 1 + 3 * NB]
    gsem = refs[4 * NB + 1 + 3 * NB:4 * NB + 1 + 4 * NB]
    ssem = refs[4 * NB + 1 + 4 * NB:4 * NB + 1 + 5 * NB]

    cid = lax.axis_index("c")
    sid = lax.axis_index("s")
    wid = sid * NC + cid
    base = wid * EPW

    zero = jnp.zeros((LANES,), jnp.float32)
    for e in range(CHUNK):
        for j in range(EMB // LANES):
            rows[0][e, pl.ds(j * LANES, LANES)] = zero
    for k in range(ROWS_PER_TILE // CHUNK):
        pltpu.sync_copy(rows[0],
                        acc.at[pl.ds(sid * ROWS_PER_TILE + k * CHUNK, CHUNK)])
    plsc.subcore_barrier()

    def fire_loads(g, b):
        off = pl.multiple_of(base + g * CHUNK, CHUNK)
        pltpu.async_copy(col_hbm.at[pl.ds(off, CHUNK)], colv[b], csem[b])
        pltpu.async_copy(row_hbm.at[pl.ds(off, CHUNK)], rowv[b], rsem[b])
        pltpu.async_copy(val_hbm.at[pl.ds(off, CHUNK)], valx[b], vsem[b])

    def fire_gather(g, b):
        off = pl.multiple_of(base + g * CHUNK, CHUNK)
        pltpu.make_async_copy(col_hbm.at[pl.ds(off, CHUNK)], colv[b],
                              csem[b]).wait()
        pltpu.async_copy(emb_hbm.at[colv[b]], rows[b], gsem[b])

    def wait_scatter(b):
        pltpu.make_async_copy(rows[b], acc.at[rowv[b]], ssem[b]).wait()

    fire_loads(0, 0)
    fire_loads(1, 1)
    fire_gather(0, 0)

    def quad_body(h, carry):
        g0 = 4 * h
        for i in range(NB):
            g = g0 + i
            bp = (i + 2) % NB
            if i < 2:
                @pl.when(h >= 1)
                def _():
                    wait_scatter(bp)
                fire_loads(g + 2, bp)
            else:
                wait_scatter(bp)

                @pl.when(h < NCHUNK // NB - 1)
                def _():
                    fire_loads(g + 2, bp)
            if i < NB - 1:
                fire_gather(g + 1, (i + 1) % NB)
            else:
                @pl.when(h < NCHUNK // NB - 1)
                def _():
                    fire_gather(g + 1, (i + 1) % NB)
            off = pl.multiple_of(base + g * CHUNK, CHUNK)
            pltpu.make_async_copy(emb_hbm.at[colv[i]], rows[i], gsem[i]).wait()
            pltpu.make_async_copy(val_hbm.at[pl.ds(off, CHUNK)], valx[i],
                                  vsem[i]).wait()
            _scale_rows(rows[i], valx[i])
            pltpu.make_async_copy(row_hbm.at[pl.ds(off, CHUNK)], rowv[i],
                                  rsem[i]).wait()
            pltpu.async_copy(rows[i], acc.at[rowv[i]], ssem[i], add=True)
        return carry

    lax.fori_loop(0, NCHUNK // NB, quad_body, 0)
    wait_scatter(2)
    wait_scatter(3)
    plsc.subcore_barrier()

    for k in range(ROWS_PER_TILE // CHUNK):
        r0 = sid * ROWS_PER_TILE + k * CHUNK
        pltpu.sync_copy(acc.at[pl.ds(r0, CHUNK)], out_hbm.at[cid, pl.ds(r0, CHUNK)])


@jax.jit
def _sc_spmm(col2, row_idx, val, embp):
    emb2m = embp.reshape(2 * NUM_ITEMS, EMB)
    mesh = plsc.VectorSubcoreMesh(core_axis_name="c", subcore_axis_name="s")
    f = pl.kernel(
        _sc_body,
        out_type=jax.ShapeDtypeStruct((NC, NUM_IDS, EMB), jnp.float32),
        mesh=mesh,
        scratch_types=(
            [pltpu.VMEM((CHUNK,), jnp.int32) for _ in range(NB)]
            + [pltpu.VMEM((CHUNK,), jnp.int32) for _ in range(NB)]
            + [pltpu.VMEM((CHUNK,), jnp.float32) for _ in range(NB)]
            + [pltpu.VMEM((CHUNK, EMB), jnp.float32) for _ in range(NB)]
            + [pltpu.VMEM_SHARED((NUM_IDS, EMB), jnp.float32)]
            + [pltpu.SemaphoreType.DMA for _ in range(5 * NB)]
        ),
        compiler_params=pltpu.CompilerParams(use_tc_tiling_on_sc=False),
    )
    return f(col2, row_idx, val, emb2m).reshape(-1)


TR = 16384


def _tp_body(t_ref, o_ref):
    y = jnp.swapaxes(t_ref[...], 0, 1)
    o_ref[...] = jnp.concatenate(
        [y, jnp.zeros((y.shape[0], EMB), jnp.float32)], axis=1)


@jax.jit
def _transpose_pad(embt):
    return pl.pallas_call(
        _tp_body,
        out_shape=jax.ShapeDtypeStruct((NUM_ITEMS, 2 * EMB), jnp.float32),
        grid=(pl.cdiv(NUM_ITEMS, TR),),
        in_specs=[pl.BlockSpec((EMB, TR), lambda i: (0, i))],
        out_specs=pl.BlockSpec((TR, 2 * EMB), lambda i: (i, 0)),
    )(embt)


def _add_body(a_ref, b_ref, o_ref):
    o_ref[...] = a_ref[...] + b_ref[...]


HALF = NUM_IDS * EMB
BLK = 65536


@jax.jit
def _combine(pflat):
    return pl.pallas_call(
        _add_body,
        out_shape=jax.ShapeDtypeStruct((HALF,), jnp.float32),
        grid=(HALF // BLK,),
        in_specs=[pl.BlockSpec((BLK,), lambda i: (i,)),
                  pl.BlockSpec((BLK,), lambda i: (i + HALF // BLK,))],
        out_specs=pl.BlockSpec((BLK,), lambda i: (i,)),
    )(pflat, pflat)


def kernel(row_idx, col_idx, data_tensor, num_ids, embeddings):
    mkey = jax.random.key(42)
    mask = jnp.floor(jax.random.uniform(mkey, data_tensor.shape) + KPROB).astype(bool)
    val = jnp.where(mask, data_tensor * (1.0 / KPROB), 0.0)
    col2 = (col_idx * 2).astype(jnp.int32)
    row32 = row_idx.astype(jnp.int32)
    embp = _transpose_pad(embeddings.T)
    pflat = _sc_spmm(col2, row32, val, embp)
    return _combine(pflat).reshape(NUM_IDS, EMB)


# speedup vs baseline: 6.3946x; 1.0126x over previous
import functools

import jax
import jax.numpy as jnp
from jax import lax
from jax.experimental import pallas as pl
from jax.experimental.pallas import tpu as pltpu
from jax.experimental.pallas import tpu_sc as plsc

NUM_IDS = 16384
NUM_ITEMS = 1000000
EMB = 64
NNZ = 819200
KPROB = 0.8

NC = 2
NS = 16
NW = NC * NS
CHUNK = 128
EPW = NNZ // NW
NCHUNK = EPW // CHUNK
NB = 4
ROWS_PER_TILE = NUM_IDS // NS
LANES = 16


def _splat(vec, i):
    idx = jnp.full((LANES,), i, jnp.int32)
    return lax.gather(
        vec, idx[:, None],
        dimension_numbers=lax.GatherDimensionNumbers(
            offset_dims=(), collapsed_slice_dims=(0,), start_index_map=(0,)),
        slice_sizes=(1,), mode=lax.GatherScatterMode.PROMISE_IN_BOUNDS)


def _scale_rows(rows, valx):
    @plsc.parallel_loop(0, CHUNK, step=LANES, unroll=2)
    def _(e0):
        vvec = valx[pl.ds(e0, LANES)]
        svs = [_splat(vvec, i) for i in range(LANES)]
        for i in range(LANES):
            for j in range(EMB // LANES):
                sl = pl.ds(j * LANES, LANES)
                rows[e0 + i, sl] = rows[e0 + i, sl] * svs[i]


def _sc_body(col_hbm, row_hbm, val_hbm, emb_hbm, out_hbm, *refs):
    colv = refs[0:NB]
    rowv = refs[NB:2 * NB]
    valx = refs[2 * NB:3 * NB]
    rows = refs[3 * NB:4 * NB]
    acc = refs[4 * NB]
    csem = refs[4 * NB + 1:4 * NB + 1 + NB]
    rsem = refs[4 * NB + 1 + NB:4 * NB + 1 + 2 * NB]
    vsem = refs[4 * NB + 1 + 2 * NB:4 * NB + 1 + 3 * NB]
    gsem = refs[4 * NB + 1 + 3 * NB:4 * NB + 1 + 4 * NB]
    ssem = refs[4 * NB + 1 + 4 * NB:4 * NB + 1 + 5 * NB]

    cid = lax.axis_index("c")
    sid = lax.axis_index("s")
    wid = sid * NC + cid
    base = wid * EPW

    zero = jnp.zeros((LANES,), jnp.float32)
    for e in range(CHUNK):
        for j in range(EMB // LANES):
            rows[0][e, pl.ds(j * LANES, LANES)] = zero
    for k in range(ROWS_PER_TILE // CHUNK):
        pltpu.sync_copy(rows[0],
                        acc.at[pl.ds(sid * ROWS_PER_TILE + k * CHUNK, CHUNK)])
    plsc.subcore_barrier()

    def fire_loads(g, b):
        off = pl.multiple_of(base + g * CHUNK, CHUNK)
        pltpu.async_copy(col_hbm.at[pl.ds(off, CHUNK)], colv[b], csem[b])
        pltpu.async_copy(row_hbm.at[pl.ds(off, CHUNK)], rowv[b], rsem[b])
        pltpu.async_copy(val_hbm.at[pl.ds(off, CHUNK)], valx[b], vsem[b])

    def fire_gather(g, b):
        off = pl.multiple_of(base + g * CHUNK, CHUNK)
        pltpu.make_async_copy(col_hbm.at[pl.ds(off, CHUNK)], colv[b],
                              csem[b]).wait()
        pltpu.async_copy(emb_hbm.at[colv[b]], rows[b], gsem[b])

    def wait_scatter(b):
        pltpu.make_async_copy(rows[b], acc.at[rowv[b]], ssem[b]).wait()

    fire_loads(0, 0)
    fire_loads(1, 1)
    fire_gather(0, 0)

    def quad_body(h, carry):
        g0 = 4 * h
        for i in range(NB):
            g = g0 + i
            bp = (i + 2) % NB
            if i < 2:
                @pl.when(h >= 1)
                def _():
                    wait_scatter(bp)
                fire_loads(g + 2, bp)
            else:
                wait_scatter(bp)

                @pl.when(h < NCHUNK // NB - 1)
                def _():
                    fire_loads(g + 2, bp)
            if i < NB - 1:
                fire_gather(g + 1, (i + 1) % NB)
            else:
                @pl.when(h < NCHUNK // NB - 1)
                def _():
                    fire_gather(g + 1, (i + 1) % NB)
            off = pl.multiple_of(base + g * CHUNK, CHUNK)
            pltpu.make_async_copy(emb_hbm.at[colv[i]], rows[i], gsem[i]).wait()
            pltpu.make_async_copy(val_hbm.at[pl.ds(off, CHUNK)], valx[i],
                                  vsem[i]).wait()
            _scale_rows(rows[i], valx[i])
            pltpu.make_async_copy(row_hbm.at[pl.ds(off, CHUNK)], rowv[i],
                                  rsem[i]).wait()
            pltpu.async_copy(rows[i], acc.at[rowv[i]], ssem[i], add=True)
        return carry

    lax.fori_loop(0, NCHUNK // NB, quad_body, 0)
    wait_scatter(2)
    wait_scatter(3)
    plsc.subcore_barrier()

    for k in range(ROWS_PER_TILE // CHUNK):
        r0 = sid * ROWS_PER_TILE + k * CHUNK
        pltpu.sync_copy(acc.at[pl.ds(r0, CHUNK)], out_hbm.at[cid, pl.ds(r0, CHUNK)])


@jax.jit
def _sc_spmm(col2, row_idx, val, embp):
    emb2m = embp.reshape(2 * NUM_ITEMS, EMB)
    mesh = plsc.VectorSubcoreMesh(core_axis_name="c", subcore_axis_name="s")
    f = pl.kernel(
        _sc_body,
        out_type=jax.ShapeDtypeStruct((NC, NUM_IDS, EMB), jnp.float32),
        mesh=mesh,
        scratch_types=(
            [pltpu.VMEM((CHUNK,), jnp.int32) for _ in range(NB)]
            + [pltpu.VMEM((CHUNK,), jnp.int32) for _ in range(NB)]
            + [pltpu.VMEM((CHUNK,), jnp.float32) for _ in range(NB)]
            + [pltpu.VMEM((CHUNK, EMB), jnp.float32) for _ in range(NB)]
            + [pltpu.VMEM_SHARED((NUM_IDS, EMB), jnp.float32)]
            + [pltpu.SemaphoreType.DMA for _ in range(5 * NB)]
        ),
        compiler_params=pltpu.CompilerParams(use_tc_tiling_on_sc=False),
    )
    return f(col2, row_idx, val, emb2m).reshape(-1)


TR = 32768


def _tp_body(t_ref, o_ref):
    y = jnp.swapaxes(t_ref[...], 0, 1)
    o_ref[...] = jnp.concatenate(
        [y, jnp.zeros((y.shape[0], EMB), jnp.float32)], axis=1)


@jax.jit
def _transpose_pad(embt):
    return pl.pallas_call(
        _tp_body,
        out_shape=jax.ShapeDtypeStruct((NUM_ITEMS, 2 * EMB), jnp.float32),
        grid=(pl.cdiv(NUM_ITEMS, TR),),
        in_specs=[pl.BlockSpec((EMB, TR), lambda i: (0, i))],
        out_specs=pl.BlockSpec((TR, 2 * EMB), lambda i: (i, 0)),
    )(embt)


def _add_body(a_ref, b_ref, o_ref):
    o_ref[...] = a_ref[...] + b_ref[...]


HALF = NUM_IDS * EMB
BLK = 65536


@jax.jit
def _combine(pflat):
    return pl.pallas_call(
        _add_body,
        out_shape=jax.ShapeDtypeStruct((HALF,), jnp.float32),
        grid=(HALF // BLK,),
        in_specs=[pl.BlockSpec((BLK,), lambda i: (i,)),
                  pl.BlockSpec((BLK,), lambda i: (i + HALF // BLK,))],
        out_specs=pl.BlockSpec((BLK,), lambda i: (i,)),
    )(pflat, pflat)


def kernel(row_idx, col_idx, data_tensor, num_ids, embeddings):
    mkey = jax.random.key(42)
    mask = jnp.floor(jax.random.uniform(mkey, data_tensor.shape) + KPROB).astype(bool)
    val = jnp.where(mask, data_tensor * (1.0 / KPROB), 0.0)
    col2 = (col_idx * 2).astype(jnp.int32)
    row32 = row_idx.astype(jnp.int32)
    embp = _transpose_pad(embeddings.T)
    pflat = _sc_spmm(col2, row32, val, embp)
    return _combine(pflat).reshape(NUM_IDS, EMB)
